# Initial kernel scaffold; baseline (speedup 1.0000x reference)
import jax
import jax.numpy as jnp
from jax.experimental import pallas as pl


def kernel(num_steps, betas):
    raise NotImplementedError("write your pallas kernel here")


# speedup vs baseline: 4.5423x; 4.5423x over previous
import functools

import jax
import jax.numpy as jnp
from jax import lax
from jax.experimental import pallas as pl
from jax.experimental.pallas import tpu as pltpu, tpu_sc as plsc

_BATCH = 16384
_TABLE = 1000
_TABLE_PAD = 1024
_LANES = 16


def _make_kernel():
    info = plsc.get_sparse_core_info()
    nc, ns = info.num_cores, info.num_subcores
    nw = nc * ns
    b_per_w = _BATCH // nw

    mesh = plsc.VectorSubcoreMesh(core_axis_name="c", subcore_axis_name="s")

    @functools.partial(
        pl.kernel,
        out_type=jax.ShapeDtypeStruct((_BATCH,), jnp.float32),
        mesh=mesh,
        scratch_types=[
            pltpu.VMEM((_TABLE_PAD,), jnp.float32),
            pltpu.VMEM((b_per_w,), jnp.int32),
            pltpu.VMEM((b_per_w,), jnp.float32),
        ],
        compiler_params=pltpu.CompilerParams(needs_layout_passes=False),
    )
    def beta_gather(idx_hbm, betas_hbm, out_hbm, table_v, idx_v, out_v):
        wid = lax.axis_index("s") * nc + lax.axis_index("c")
        base = wid * b_per_w
        pltpu.sync_copy(betas_hbm, table_v)
        pltpu.sync_copy(idx_hbm.at[pl.ds(base, b_per_w)], idx_v)
        for i in range(b_per_w // _LANES):
            ids = idx_v[pl.ds(i * _LANES, _LANES)]
            out_v[pl.ds(i * _LANES, _LANES)] = plsc.load_gather(table_v, [ids])
        pltpu.sync_copy(out_v, out_hbm.at[pl.ds(base, b_per_w)])

    return beta_gather


_beta_gather = _make_kernel()


@jax.jit
def kernel(num_steps, betas):
    betas_padded = jnp.pad(betas, (0, _TABLE_PAD - _TABLE))
    out = _beta_gather(num_steps, betas_padded)
    return out.reshape((_BATCH, 1))


# speedup vs baseline: 4.6635x; 1.0267x over previous
import functools

import jax
import jax.numpy as jnp
from jax import lax
from jax.experimental import pallas as pl
from jax.experimental.pallas import tpu as pltpu, tpu_sc as plsc

_BATCH = 16384
_TABLE = 1000
_TABLE_PAD = 1024
_LANES = 16


def _make_kernel():
    info = plsc.get_sparse_core_info()
    nc, ns = info.num_cores, info.num_subcores
    nw = nc * ns
    b_per_w = _BATCH // nw

    mesh = plsc.VectorSubcoreMesh(core_axis_name="c", subcore_axis_name="s")

    @functools.partial(
        pl.kernel,
        out_type=jax.ShapeDtypeStruct((_BATCH,), jnp.float32),
        mesh=mesh,
        scratch_types=[
            pltpu.VMEM((_TABLE,), jnp.float32),
            pltpu.VMEM((b_per_w,), jnp.int32),
            pltpu.VMEM((b_per_w,), jnp.float32),
            pltpu.SemaphoreType.DMA,
            pltpu.SemaphoreType.DMA,
        ],
        compiler_params=pltpu.CompilerParams(needs_layout_passes=False),
    )
    def beta_gather(idx_hbm, betas_hbm, out_hbm, table_v, idx_v, out_v, sem_t, sem_i):
        wid = lax.axis_index("s") * nc + lax.axis_index("c")
        base = wid * b_per_w
        cp_t = pltpu.async_copy(betas_hbm, table_v, sem_t)
        cp_i = pltpu.async_copy(idx_hbm.at[pl.ds(base, b_per_w)], idx_v, sem_i)
        cp_t.wait()
        cp_i.wait()
        for i in range(b_per_w // _LANES):
            ids = idx_v[pl.ds(i * _LANES, _LANES)]
            out_v[pl.ds(i * _LANES, _LANES)] = plsc.load_gather(table_v, [ids])
        pltpu.sync_copy(out_v, out_hbm.at[pl.ds(base, b_per_w)])

    return beta_gather


_beta_gather = _make_kernel()


@jax.jit
def kernel(num_steps, betas):
    out = _beta_gather(num_steps, betas)
    return out.reshape((_BATCH, 1))


# speedup vs baseline: 5.0367x; 1.0800x over previous
import functools

import jax
import jax.numpy as jnp
from jax import lax
from jax.experimental import pallas as pl
from jax.experimental.pallas import tpu as pltpu, tpu_sc as plsc

_BATCH = 16384
_TABLE = 1000
_TABLE_PAD = 1024
_LANES = 16


def _make_kernel():
    info = plsc.get_sparse_core_info()
    nc, ns = 1, info.num_subcores
    nw = nc * ns
    b_per_w = _BATCH // nw

    mesh = plsc.VectorSubcoreMesh(
        core_axis_name="c", subcore_axis_name="s", num_cores=nc
    )

    @functools.partial(
        pl.kernel,
        out_type=jax.ShapeDtypeStruct((_BATCH,), jnp.float32),
        mesh=mesh,
        scratch_types=[
            pltpu.VMEM((_TABLE,), jnp.float32),
            pltpu.VMEM((b_per_w,), jnp.int32),
            pltpu.VMEM((b_per_w,), jnp.float32),
            pltpu.SemaphoreType.DMA,
            pltpu.SemaphoreType.DMA,
        ],
        compiler_params=pltpu.CompilerParams(needs_layout_passes=False),
    )
    def beta_gather(idx_hbm, betas_hbm, out_hbm, table_v, idx_v, out_v, sem_t, sem_i):
        wid = lax.axis_index("s") * nc + lax.axis_index("c")
        base = wid * b_per_w
        cp_t = pltpu.async_copy(betas_hbm, table_v, sem_t)
        cp_i = pltpu.async_copy(idx_hbm.at[pl.ds(base, b_per_w)], idx_v, sem_i)
        cp_t.wait()
        cp_i.wait()
        for i in range(b_per_w // _LANES):
            ids = idx_v[pl.ds(i * _LANES, _LANES)]
            out_v[pl.ds(i * _LANES, _LANES)] = plsc.load_gather(table_v, [ids])
        pltpu.sync_copy(out_v, out_hbm.at[pl.ds(base, b_per_w)])

    return beta_gather


_beta_gather = _make_kernel()


@jax.jit
def kernel(num_steps, betas):
    out = _beta_gather(num_steps, betas)
    return out.reshape((_BATCH, 1))


# speedup vs baseline: 5.0438x; 1.0014x over previous
import functools

import jax
import jax.numpy as jnp
from jax import lax
from jax.experimental import pallas as pl
from jax.experimental.pallas import tpu as pltpu, tpu_sc as plsc

_BATCH = 16384
_TABLE = 1000
_TABLE_PAD = 1024
_LANES = 16


def _make_kernel():
    info = plsc.get_sparse_core_info()
    nc, ns = 1, info.num_subcores
    nw = nc * ns
    b_per_w = _BATCH // nw

    mesh = plsc.VectorSubcoreMesh(
        core_axis_name="c", subcore_axis_name="s", num_cores=nc
    )

    @functools.partial(
        pl.kernel,
        out_type=jax.ShapeDtypeStruct((_BATCH,), jnp.float32),
        mesh=mesh,
        scratch_types=[
            pltpu.VMEM((_TABLE,), jnp.float32),
            pltpu.VMEM((b_per_w,), jnp.int32),
            pltpu.VMEM((b_per_w,), jnp.float32),
            pltpu.SemaphoreType.DMA,
            pltpu.SemaphoreType.DMA,
        ],
        compiler_params=pltpu.CompilerParams(
            needs_layout_passes=False,
            disable_bounds_checks=True,
            disable_semaphore_checks=True,
            skip_device_barrier=True,
        ),
    )
    def beta_gather(idx_hbm, betas_hbm, out_hbm, table_v, idx_v, out_v, sem_t, sem_i):
        wid = lax.axis_index("s") * nc + lax.axis_index("c")
        base = wid * b_per_w
        cp_t = pltpu.async_copy(betas_hbm, table_v, sem_t)
        cp_i = pltpu.async_copy(idx_hbm.at[pl.ds(base, b_per_w)], idx_v, sem_i)
        cp_t.wait()
        cp_i.wait()
        for i in range(b_per_w // _LANES):
            ids = idx_v[pl.ds(i * _LANES, _LANES)]
            out_v[pl.ds(i * _LANES, _LANES)] = plsc.load_gather(table_v, [ids])
        pltpu.sync_copy(out_v, out_hbm.at[pl.ds(base, b_per_w)])

    return beta_gather


_beta_gather = _make_kernel()


@jax.jit
def kernel(num_steps, betas):
    out = _beta_gather(num_steps, betas)
    return out.reshape((_BATCH, 1))


# speedup vs baseline: 5.0648x; 1.0041x over previous
import functools

import jax
import jax.numpy as jnp
from jax import lax
from jax.experimental import pallas as pl
from jax.experimental.pallas import tpu as pltpu, tpu_sc as plsc

_BATCH = 16384
_TABLE = 1000
_TABLE_PAD = 1024
_LANES = 16


def _make_kernel():
    info = plsc.get_sparse_core_info()
    nc, ns = 1, info.num_subcores
    nw = nc * ns
    b_per_w = _BATCH // nw

    mesh = plsc.VectorSubcoreMesh(
        core_axis_name="c", subcore_axis_name="s", num_cores=nc
    )

    half = b_per_w // 2

    @functools.partial(
        pl.kernel,
        out_type=jax.ShapeDtypeStruct((_BATCH,), jnp.float32),
        mesh=mesh,
        scratch_types=[
            pltpu.VMEM((_TABLE,), jnp.float32),
            pltpu.VMEM((b_per_w,), jnp.int32),
            pltpu.VMEM((b_per_w,), jnp.float32),
            pltpu.SemaphoreType.DMA,
            pltpu.SemaphoreType.DMA,
            pltpu.SemaphoreType.DMA,
            pltpu.SemaphoreType.DMA,
        ],
        compiler_params=pltpu.CompilerParams(needs_layout_passes=False),
    )
    def beta_gather(
        idx_hbm, betas_hbm, out_hbm, table_v, idx_v, out_v, sem_t, sem_i0, sem_i1, sem_o
    ):
        wid = lax.axis_index("s") * nc + lax.axis_index("c")
        base = wid * b_per_w
        cp_t = pltpu.async_copy(betas_hbm, table_v, sem_t)
        cp_i0 = pltpu.async_copy(
            idx_hbm.at[pl.ds(base, half)], idx_v.at[pl.ds(0, half)], sem_i0
        )
        cp_i1 = pltpu.async_copy(
            idx_hbm.at[pl.ds(base + half, half)], idx_v.at[pl.ds(half, half)], sem_i1
        )
        cp_t.wait()
        cp_i0.wait()
        for i in range(half // _LANES):
            ids = idx_v[pl.ds(i * _LANES, _LANES)]
            out_v[pl.ds(i * _LANES, _LANES)] = plsc.load_gather(table_v, [ids])
        cp_o0 = pltpu.async_copy(
            out_v.at[pl.ds(0, half)], out_hbm.at[pl.ds(base, half)], sem_o
        )
        cp_i1.wait()
        for i in range(half // _LANES, b_per_w // _LANES):
            ids = idx_v[pl.ds(i * _LANES, _LANES)]
            out_v[pl.ds(i * _LANES, _LANES)] = plsc.load_gather(table_v, [ids])
        cp_o1 = pltpu.async_copy(
            out_v.at[pl.ds(half, half)], out_hbm.at[pl.ds(base + half, half)], sem_o
        )
        cp_o0.wait()
        cp_o1.wait()

    return beta_gather


_beta_gather = _make_kernel()


@jax.jit
def kernel(num_steps, betas):
    out = _beta_gather(num_steps, betas)
    return out.reshape((_BATCH, 1))
